# baseline (speedup 1.0000x reference)
import functools

import jax
import jax.numpy as jnp
from jax import lax
from jax.experimental import pallas as pl
from jax.experimental.pallas import tpu as pltpu
from jax.experimental.pallas import tpu_sc as plsc

N = 10000
E = 320000
C = 128
H = 4

NT = 16
NP = 10240
NS = NP // NT
SG = NS // 16
EPT = E // NT
CHUNK = 250
NCHUNK = EPT // (CHUNK * 16)

_SC_PARAMS = pltpu.CompilerParams(needs_layout_passes=False)
_MESH = plsc.VectorSubcoreMesh(core_axis_name="c", subcore_axis_name="s",
                               num_cores=1)


def _rsqrt_newton(x):
    i = plsc.bitcast(x, jnp.int32)
    i = jnp.int32(0x5F3759DF) - lax.shift_right_arithmetic(i, 1)
    y = plsc.bitcast(i, jnp.float32)
    for _ in range(3):
        y = y * (1.5 - 0.5 * x * y * y)
    return y


@functools.partial(
    pl.kernel,
    out_type=jax.ShapeDtypeStruct((1, NP), jnp.float32),
    mesh=_MESH,
    compiler_params=_SC_PARAMS,
    scratch_types=[pltpu.VMEM((CHUNK * 16,), jnp.int32),
                   pltpu.VMEM((CHUNK * 16,), jnp.int32),
                   pltpu.VMEM((CHUNK * 16,), jnp.int32),
                   pltpu.VMEM((CHUNK * 16,), jnp.int32),
                   pltpu.VMEM((CHUNK * 16,), jnp.int32),
                   pltpu.VMEM((CHUNK * 16,), jnp.int32),
                   pltpu.VMEM((1, H * NP), jnp.float32),
                   pltpu.VMEM((1, H * NP), jnp.float32),
                   pltpu.VMEM((1, 6 * NS), jnp.float32),
                   pltpu.VMEM((NS,), jnp.float32),
                   pltpu.VMEM((1, 144), jnp.float32),
                   pltpu.VMEM((1,), jnp.int32),
                   pltpu.VMEM_SHARED((1, NP), jnp.float32),
                   pltpu.VMEM_SHARED((1, H * NP), jnp.float32),
                   pltpu.VMEM_SHARED((1, H * NP), jnp.float32),
                   pltpu.SemaphoreType.DMA,
                   pltpu.SemaphoreType.DMA,
                   pltpu.SemaphoreType.DMA,
                   pltpu.SemaphoreType.DMA,
                   pltpu.SemaphoreType.DMA,
                   pltpu.SemaphoreType.DMA,
                   pltpu.SemaphoreType.DMA,
                   pltpu.SemaphoreType.DMA,
                   pltpu.SemaphoreType.DMA,
                   pltpu.SemaphoreType.DMA],
)
def _sc_gcn(xw_hbm, ei_hbm, z4_hbm, zn_hbm, params_hbm, zi_hbm,
            out_hbm,
            src_v0, src_v1, src_v2, dst_v0, dst_v1, dst_v2,
            q_v, acc_v, sl_v, dinv_v, par_v, idx_v,
            shn, sh4, shq,
            semA, semD, semE, semF,
            semS0, semS1, semS2, semD0, semD1, semD2):
    t = lax.axis_index("s")
    n0 = t * NS
    ebase = t * EPT
    ones16 = jnp.ones((16,), jnp.float32)
    ssems = (semS0, semS1, semS2)
    dsems = (semD0, semD1, semD2)
    sbufs = (src_v0, src_v1, src_v2)
    dbufs = (dst_v0, dst_v1, dst_v2)
    qf = q_v.at[0]
    accf = acc_v.at[0]
    slf = sl_v.at[0]
    parf = par_v.at[0]

    def edge_stream(body, with_src):
        cps = [None, None, None]

        def fire(ci):
            b = ci % 3
            o0 = ebase + ci * CHUNK * 16
            cpd = pltpu.async_copy(ei_hbm.at[pl.ds(E + o0, CHUNK * 16)],
                                   dbufs[b], dsems[b])
            cps_ = cpd
            if with_src:
                cps_ = (pltpu.async_copy(ei_hbm.at[pl.ds(o0, CHUNK * 16)],
                                         sbufs[b], ssems[b]), cpd)
            cps[b] = cps_

        fire(0)
        fire(1)
        for ci in range(NCHUNK):
            if ci + 2 < NCHUNK:
                fire(ci + 2)
            got = cps[ci % 3]
            if with_src:
                got[0].wait()
                got[1].wait()
            else:
                got.wait()
            body(sbufs[ci % 3], dbufs[ci % 3])

    cpA = pltpu.async_copy(xw_hbm, q_v, semA)
    cpD = pltpu.async_copy(z4_hbm, acc_v, semD)
    cpE = pltpu.async_copy(params_hbm, par_v, semE)
    cpF = pltpu.async_copy(zi_hbm, idx_v, semF)

    @pl.when(t == 0)
    def _():
        pltpu.sync_copy(zn_hbm, shn)
        pltpu.sync_copy(z4_hbm, sh4)

    plsc.subcore_barrier()

    with jax.named_scope("phA_zwait"):
        cpD.wait()
        cpF.wait()

    def deg_body(_sbuf, dbuf):
        @plsc.parallel_loop(0, CHUNK, 1, unroll=5)
        def _(i):
            d = dbuf[pl.ds(i * 16, 16)]
            plsc.addupdate_scatter(accf, [d], ones16)

    with jax.named_scope("phA_deg"):
        edge_stream(deg_body, with_src=False)

    with jax.named_scope("phA_red"):
        pltpu.sync_copy(acc_v.at[:, pl.ds(0, NP)], shn.at[idx_v],
                        add=True)
    cpD2 = pltpu.async_copy(z4_hbm, acc_v, semD)
    plsc.subcore_barrier()

    with jax.named_scope("phA2_slice"):
        pltpu.sync_copy(shn.at[:, pl.ds(n0, NS)],
                        sl_v.at[:, pl.ds(5 * NS, NS)])
    with jax.named_scope("phA2_xwwait"):
        cpA.wait()
        cpE.wait()
    def a2_body(g, c):
        o = g * 16
        deg = slf[pl.ds(5 * NS + o, 16)] + 1.0
        dv = _rsqrt_newton(deg)
        dinv_v[pl.ds(o, 16)] = dv
        for j in range(H):
            qi = pl.ds(j * NP + n0 + o, 16)
            qf[qi] = dv * qf[qi]
        return c

    with jax.named_scope("phA2_loop"):
        lax.fori_loop(0, SG, a2_body, 0)
    with jax.named_scope("phA2_pub"):
        for j in range(H):
            pltpu.sync_copy(q_v.at[:, pl.ds(j * NP + n0, NS)],
                            shq.at[:, pl.ds(j * NP + n0, NS)])
    with jax.named_scope("bar3"):
        plsc.subcore_barrier()

    with jax.named_scope("phB_q1rd"):
        pltpu.sync_copy(shq, q_v)
    cpD2.wait()

    def agg4_body(sbuf, dbuf):
        @plsc.parallel_loop(0, CHUNK, 1, unroll=4)
        def _(i):
            s = sbuf[pl.ds(i * 16, 16)]
            d = dbuf[pl.ds(i * 16, 16)]
            for j in range(H):
                si = s if j == 0 else s + (j * NP)
                di = d if j == 0 else d + (j * NP)
                g = plsc.load_gather(qf, [si])
                plsc.addupdate_scatter(accf, [di], g)

    with jax.named_scope("phB_edges"):
        edge_stream(agg4_body, with_src=True)

    with jax.named_scope("phB_red"):
        pltpu.sync_copy(acc_v, sh4.at[idx_v], add=True)

    @pl.when(t == 0)
    def _():
        pltpu.sync_copy(zn_hbm, shn)

    plsc.subcore_barrier()

    with jax.named_scope("phB2_slice"):
        for j in range(H):
            pltpu.sync_copy(sh4.at[:, pl.ds(j * NP + n0, NS)],
                            sl_v.at[:, pl.ds(j * NS, NS)])
    cpD3 = pltpu.async_copy(z4_hbm, acc_v, semD)

    def b2_body(g, c):
        o = g * 16
        dv = dinv_v[pl.ds(o, 16)]
        hw = jnp.zeros((16,), jnp.float32)
        for j in range(H):
            aj = slf[pl.ds(j * NS + o, 16)] + qf[pl.ds(j * NP + n0 + o, 16)]
            hj = jnp.maximum(dv * aj + parf[pl.ds(j * 16, 16)], 0.0)
            hw = hw + hj * parf[pl.ds((4 + j) * 16, 16)]
        slf[pl.ds(4 * NS + o, 16)] = dv * hw
        return c

    with jax.named_scope("phB2_loop"):
        lax.fori_loop(0, SG, b2_body, 0)
    with jax.named_scope("phB2_pub"):
        pltpu.sync_copy(sl_v.at[:, pl.ds(4 * NS, NS)],
                        shq.at[:, pl.ds(n0, NS)])
    with jax.named_scope("bar5"):
        plsc.subcore_barrier()

    with jax.named_scope("phC_q2rd"):
        pltpu.sync_copy(shq.at[:, pl.ds(0, NP)], q_v.at[:, pl.ds(0, NP)])
    with jax.named_scope("phC_zwait"):
        cpD3.wait()

    def agg1_body(sbuf, dbuf):
        @plsc.parallel_loop(0, CHUNK, 1, unroll=5)
        def _(i):
            s = sbuf[pl.ds(i * 16, 16)]
            d = dbuf[pl.ds(i * 16, 16)]
            g = plsc.load_gather(qf, [s])
            plsc.addupdate_scatter(accf, [d], g)

    edge_stream(agg1_body, with_src=True)

    pltpu.sync_copy(acc_v.at[:, pl.ds(0, NP)], shn.at[idx_v], add=True)
    plsc.subcore_barrier()

    pltpu.sync_copy(shn.at[:, pl.ds(n0, NS)],
                    sl_v.at[:, pl.ds(5 * NS, NS)])
    def c2_body(g, c):
        o = g * 16
        dv = dinv_v[pl.ds(o, 16)]
        z = (dv * (slf[pl.ds(5 * NS + o, 16)] + slf[pl.ds(4 * NS + o, 16)])
             + parf[pl.ds(8 * 16, 16)])
        slf[pl.ds(3 * NS + o, 16)] = 1.0 / (1.0 + jnp.exp(-z))
        return c

    lax.fori_loop(0, SG, c2_body, 0)
    pltpu.sync_copy(sl_v.at[:, pl.ds(3 * NS, NS)],
                    out_hbm.at[:, pl.ds(n0, NS)])


def _tc0_body(x_ref, w1_ref, xwt_ref):
    xwt_ref[...] = jnp.zeros((H, NP), jnp.float32)
    xwt_ref[:, :N] = lax.dot_general(w1_ref[...], x_ref[...],
                                     (((1,), (1,)), ((), ())),
                                     preferred_element_type=jnp.float32)


_tc0 = pl.pallas_call(
    _tc0_body,
    out_shape=jax.ShapeDtypeStruct((H, NP), jnp.float32))


def kernel(x, edge_index, W1, b1, W2, b2):
    ei = edge_index.astype(jnp.int32).reshape(2 * E)
    xwt = _tc0(x, W1)

    z4 = jnp.zeros((1, H * NP), jnp.float32)
    zn = jnp.zeros((1, NP), jnp.float32)
    zi = jnp.zeros((1,), jnp.int32)
    params = jnp.concatenate(
        [jnp.broadcast_to(b1.reshape(H, 1), (H, 16)),
         jnp.broadcast_to(W2.reshape(H, 1), (H, 16)),
         jnp.broadcast_to(b2.reshape(1, 1), (1, 16))],
        axis=0).reshape(1, 144)

    out_pad = _sc_gcn(xwt.reshape(1, H * NP), ei,
                      z4, zn, params, zi)
    return out_pad[0, :N].reshape(N, 1)


# speedup vs baseline: 1.0576x; 1.0576x over previous
import functools

import jax
import jax.numpy as jnp
from jax import lax
from jax.experimental import pallas as pl
from jax.experimental.pallas import tpu as pltpu
from jax.experimental.pallas import tpu_sc as plsc

N = 10000
E = 320000
C = 128
H = 4

NT = 16
NP = 10240
NS = NP // NT
SG = NS // 16
EPT = E // NT
CHUNK = 250
NCHUNK = EPT // (CHUNK * 16)

_SC_PARAMS = pltpu.CompilerParams(needs_layout_passes=False)
_MESH = plsc.VectorSubcoreMesh(core_axis_name="c", subcore_axis_name="s",
                               num_cores=1)


def _rsqrt_newton(x):
    i = plsc.bitcast(x, jnp.int32)
    i = jnp.int32(0x5F3759DF) - lax.shift_right_arithmetic(i, 1)
    y = plsc.bitcast(i, jnp.float32)
    for _ in range(3):
        y = y * (1.5 - 0.5 * x * y * y)
    return y


@functools.partial(
    pl.kernel,
    out_type=jax.ShapeDtypeStruct((1, NP), jnp.float32),
    mesh=_MESH,
    compiler_params=_SC_PARAMS,
    scratch_types=[pltpu.VMEM((CHUNK * 16,), jnp.int32),
                   pltpu.VMEM((CHUNK * 16,), jnp.int32),
                   pltpu.VMEM((CHUNK * 16,), jnp.int32),
                   pltpu.VMEM((CHUNK * 16,), jnp.int32),
                   pltpu.VMEM((CHUNK * 16,), jnp.int32),
                   pltpu.VMEM((CHUNK * 16,), jnp.int32),
                   pltpu.VMEM((1, H * NP), jnp.float32),
                   pltpu.VMEM((1, H * NP), jnp.float32),
                   pltpu.VMEM((1, 6 * NS), jnp.float32),
                   pltpu.VMEM((NS,), jnp.float32),
                   pltpu.VMEM((1, 144), jnp.float32),
                   pltpu.VMEM((1,), jnp.int32),
                   pltpu.VMEM_SHARED((1, NP), jnp.float32),
                   pltpu.VMEM_SHARED((1, H * NP), jnp.float32),
                   pltpu.VMEM_SHARED((1, H * NP), jnp.float32),
                   pltpu.SemaphoreType.DMA,
                   pltpu.SemaphoreType.DMA,
                   pltpu.SemaphoreType.DMA,
                   pltpu.SemaphoreType.DMA,
                   pltpu.SemaphoreType.DMA,
                   pltpu.SemaphoreType.DMA,
                   pltpu.SemaphoreType.DMA,
                   pltpu.SemaphoreType.DMA,
                   pltpu.SemaphoreType.DMA,
                   pltpu.SemaphoreType.DMA],
)
def _sc_gcn(xw_hbm, ei_hbm, z4_hbm, zn_hbm, params_hbm, zi_hbm,
            out_hbm,
            src_v0, src_v1, src_v2, dst_v0, dst_v1, dst_v2,
            q_v, acc_v, sl_v, dinv_v, par_v, idx_v,
            shn, sh4, shq,
            semA, semD, semE, semF,
            semS0, semS1, semS2, semD0, semD1, semD2):
    t = lax.axis_index("s")
    n0 = t * NS
    ebase = t * EPT
    ones16 = jnp.ones((16,), jnp.float32)
    ssems = (semS0, semS1, semS2)
    dsems = (semD0, semD1, semD2)
    sbufs = (src_v0, src_v1, src_v2)
    dbufs = (dst_v0, dst_v1, dst_v2)
    qf = q_v.at[0]
    accf = acc_v.at[0]
    slf = sl_v.at[0]
    parf = par_v.at[0]

    def edge_stream(body, with_src):
        cps = [None, None, None]

        def fire(ci):
            b = ci % 3
            o0 = ebase + ci * CHUNK * 16
            cpd = pltpu.async_copy(ei_hbm.at[pl.ds(E + o0, CHUNK * 16)],
                                   dbufs[b], dsems[b])
            cps_ = cpd
            if with_src:
                cps_ = (pltpu.async_copy(ei_hbm.at[pl.ds(o0, CHUNK * 16)],
                                         sbufs[b], ssems[b]), cpd)
            cps[b] = cps_

        fire(0)
        fire(1)
        for ci in range(NCHUNK):
            if ci + 2 < NCHUNK:
                fire(ci + 2)
            got = cps[ci % 3]
            if with_src:
                got[0].wait()
                got[1].wait()
            else:
                got.wait()
            body(sbufs[ci % 3], dbufs[ci % 3])

    cpA = pltpu.async_copy(xw_hbm, q_v, semA)
    cpD = pltpu.async_copy(z4_hbm.at[:, pl.ds(0, NP)],
                           acc_v.at[:, pl.ds(0, NP)], semD)
    cpE = pltpu.async_copy(params_hbm, par_v, semE)
    cpF = pltpu.async_copy(zi_hbm, idx_v, semF)

    @pl.when(t == 0)
    def _():
        pltpu.sync_copy(zn_hbm, shn)
        pltpu.sync_copy(z4_hbm, sh4)

    plsc.subcore_barrier()

    with jax.named_scope("phA_zwait"):
        cpD.wait()
        cpF.wait()

    def deg_body(_sbuf, dbuf):
        @plsc.parallel_loop(0, CHUNK, 1, unroll=5)
        def _(i):
            d = dbuf[pl.ds(i * 16, 16)]
            plsc.addupdate_scatter(accf, [d], ones16)

    with jax.named_scope("phA_deg"):
        edge_stream(deg_body, with_src=False)

    with jax.named_scope("phA_red"):
        pltpu.sync_copy(acc_v.at[:, pl.ds(0, NP)], shn.at[idx_v],
                        add=True)
    cpD2 = pltpu.async_copy(z4_hbm, acc_v, semD)
    plsc.subcore_barrier()

    with jax.named_scope("phA2_slice"):
        pltpu.sync_copy(shn.at[:, pl.ds(n0, NS)],
                        sl_v.at[:, pl.ds(5 * NS, NS)])
    with jax.named_scope("phA2_xwwait"):
        cpA.wait()
        cpE.wait()
    def a2_body(g, c):
        o = g * 16
        deg = slf[pl.ds(5 * NS + o, 16)] + 1.0
        dv = _rsqrt_newton(deg)
        dinv_v[pl.ds(o, 16)] = dv
        for j in range(H):
            qi = pl.ds(j * NP + n0 + o, 16)
            qf[qi] = dv * qf[qi]
        return c

    with jax.named_scope("phA2_loop"):
        lax.fori_loop(0, SG, a2_body, 0)
    with jax.named_scope("phA2_pub"):
        for j in range(H):
            pltpu.sync_copy(q_v.at[:, pl.ds(j * NP + n0, NS)],
                            shq.at[:, pl.ds(j * NP + n0, NS)])
    with jax.named_scope("bar3"):
        plsc.subcore_barrier()

    with jax.named_scope("phB_q1rd"):
        pltpu.sync_copy(shq, q_v)
    cpD2.wait()

    def agg4_body(sbuf, dbuf):
        @plsc.parallel_loop(0, CHUNK, 1, unroll=4)
        def _(i):
            s = sbuf[pl.ds(i * 16, 16)]
            d = dbuf[pl.ds(i * 16, 16)]
            for j in range(H):
                si = s if j == 0 else s + (j * NP)
                di = d if j == 0 else d + (j * NP)
                g = plsc.load_gather(qf, [si])
                plsc.addupdate_scatter(accf, [di], g)

    with jax.named_scope("phB_edges"):
        edge_stream(agg4_body, with_src=True)

    with jax.named_scope("phB_red"):
        pltpu.sync_copy(acc_v, sh4.at[idx_v], add=True)

    @pl.when(t == 0)
    def _():
        pltpu.sync_copy(zn_hbm, shn)

    plsc.subcore_barrier()

    with jax.named_scope("phB2_slice"):
        for j in range(H):
            pltpu.sync_copy(sh4.at[:, pl.ds(j * NP + n0, NS)],
                            sl_v.at[:, pl.ds(j * NS, NS)])
    cpD3 = pltpu.async_copy(z4_hbm.at[:, pl.ds(0, NP)],
                            acc_v.at[:, pl.ds(0, NP)], semD)

    def b2_body(g, c):
        o = g * 16
        dv = dinv_v[pl.ds(o, 16)]
        hw = jnp.zeros((16,), jnp.float32)
        for j in range(H):
            aj = slf[pl.ds(j * NS + o, 16)] + qf[pl.ds(j * NP + n0 + o, 16)]
            hj = jnp.maximum(dv * aj + parf[pl.ds(j * 16, 16)], 0.0)
            hw = hw + hj * parf[pl.ds((4 + j) * 16, 16)]
        slf[pl.ds(4 * NS + o, 16)] = dv * hw
        return c

    with jax.named_scope("phB2_loop"):
        lax.fori_loop(0, SG, b2_body, 0)
    with jax.named_scope("phB2_pub"):
        pltpu.sync_copy(sl_v.at[:, pl.ds(4 * NS, NS)],
                        shq.at[:, pl.ds(n0, NS)])
    with jax.named_scope("bar5"):
        plsc.subcore_barrier()

    with jax.named_scope("phC_q2rd"):
        pltpu.sync_copy(shq.at[:, pl.ds(0, NP)], q_v.at[:, pl.ds(0, NP)])
    with jax.named_scope("phC_zwait"):
        cpD3.wait()

    def agg1_body(sbuf, dbuf):
        @plsc.parallel_loop(0, CHUNK, 1, unroll=5)
        def _(i):
            s = sbuf[pl.ds(i * 16, 16)]
            d = dbuf[pl.ds(i * 16, 16)]
            g = plsc.load_gather(qf, [s])
            plsc.addupdate_scatter(accf, [d], g)

    edge_stream(agg1_body, with_src=True)

    pltpu.sync_copy(acc_v.at[:, pl.ds(0, NP)], shn.at[idx_v], add=True)
    plsc.subcore_barrier()

    pltpu.sync_copy(shn.at[:, pl.ds(n0, NS)],
                    sl_v.at[:, pl.ds(5 * NS, NS)])
    def c2_body(g, c):
        o = g * 16
        dv = dinv_v[pl.ds(o, 16)]
        z = (dv * (slf[pl.ds(5 * NS + o, 16)] + slf[pl.ds(4 * NS + o, 16)])
             + parf[pl.ds(8 * 16, 16)])
        slf[pl.ds(3 * NS + o, 16)] = 1.0 / (1.0 + jnp.exp(-z))
        return c

    lax.fori_loop(0, SG, c2_body, 0)
    pltpu.sync_copy(sl_v.at[:, pl.ds(3 * NS, NS)],
                    out_hbm.at[:, pl.ds(n0, NS)])


def _tc0_body(x_ref, w1_ref, xwt_ref):
    xwt_ref[...] = jnp.zeros((H, NP), jnp.float32)
    xwt_ref[:, :N] = lax.dot_general(w1_ref[...], x_ref[...],
                                     (((1,), (1,)), ((), ())),
                                     preferred_element_type=jnp.float32)


_tc0 = pl.pallas_call(
    _tc0_body,
    out_shape=jax.ShapeDtypeStruct((H, NP), jnp.float32))


def kernel(x, edge_index, W1, b1, W2, b2):
    ei = edge_index.astype(jnp.int32).reshape(2 * E)
    xwt = _tc0(x, W1)

    z4 = jnp.zeros((1, H * NP), jnp.float32)
    zn = jnp.zeros((1, NP), jnp.float32)
    zi = jnp.zeros((1,), jnp.int32)
    params = jnp.concatenate(
        [jnp.broadcast_to(b1.reshape(H, 1), (H, 16)),
         jnp.broadcast_to(W2.reshape(H, 1), (H, 16)),
         jnp.broadcast_to(b2.reshape(1, 1), (1, 16))],
        axis=0).reshape(1, 144)

    out_pad = _sc_gcn(xwt.reshape(1, H * NP), ei,
                      z4, zn, params, zi)
    return out_pad[0, :N].reshape(N, 1)
